# Initial kernel scaffold; baseline (speedup 1.0000x reference)
import jax
import jax.numpy as jnp
from jax.experimental import pallas as pl


def kernel(x, W):
    raise NotImplementedError("write your pallas kernel here")


# speedup vs baseline: 2.4907x; 2.4907x over previous
import functools

import jax
import jax.numpy as jnp
from jax import lax
from jax.experimental import pallas as pl
from jax.experimental.pallas import tpu as pltpu
from jax.experimental.pallas import tpu_sc as plsc

BATCH = 16384
HIST = 200
SIZE = 16
B = BATCH * HIST
NC = 2
NS = 16
NW = NC * NS
BPW = B // NW
CHUNK = 2048
NCHUNK = BPW // CHUNK


def _make_gather():
    mesh = plsc.VectorSubcoreMesh(core_axis_name="c", subcore_axis_name="s")

    @functools.partial(
        pl.kernel,
        mesh=mesh,
        out_type=jax.ShapeDtypeStruct((B, SIZE), jnp.float32),
        scratch_types=[
            pltpu.VMEM((CHUNK,), jnp.int32),
            pltpu.VMEM((CHUNK, SIZE), jnp.float32),
            pltpu.SemaphoreType.DMA,
        ],
        compiler_params=pltpu.CompilerParams(use_tc_tiling_on_sc=False),
    )
    def gather_kernel(idx_hbm, table_hbm, out_hbm, idx_v, rows_v, sem):
        wid = lax.axis_index("s") * NC + lax.axis_index("c")
        base = wid * BPW

        @pl.loop(0, NCHUNK)
        def _chunk(g):
            off = base + g * CHUNK
            pltpu.sync_copy(idx_hbm.at[pl.ds(off, CHUNK)], idx_v)
            pltpu.async_copy(table_hbm.at[idx_v], rows_v, sem).wait()
            pltpu.sync_copy(rows_v, out_hbm.at[pl.ds(off, CHUNK)])

    return gather_kernel


_gather = _make_gather()


@jax.jit
def kernel(x, W):
    xf = x.reshape(-1).astype(jnp.int32)
    out = _gather(xf, W)
    return out.reshape(BATCH, HIST, SIZE)


# speedup vs baseline: 2.5693x; 1.0315x over previous
import functools

import jax
import jax.numpy as jnp
from jax import lax
from jax.experimental import pallas as pl
from jax.experimental.pallas import tpu as pltpu
from jax.experimental.pallas import tpu_sc as plsc

BATCH = 16384
HIST = 200
SIZE = 16
B = BATCH * HIST
NC = 2
NS = 16
NW = NC * NS
BPW = B // NW
CHUNK = 2048
NCHUNK = BPW // CHUNK
NBUF = 2


def _make_gather():
    mesh = plsc.VectorSubcoreMesh(core_axis_name="c", subcore_axis_name="s")

    @functools.partial(
        pl.kernel,
        mesh=mesh,
        out_type=jax.ShapeDtypeStruct((B, SIZE), jnp.float32),
        scratch_types=[
            pltpu.VMEM((NBUF, CHUNK), jnp.int32),
            pltpu.VMEM((NBUF, CHUNK, SIZE), jnp.float32),
            [pltpu.SemaphoreType.DMA] * NBUF,
            [pltpu.SemaphoreType.DMA] * NBUF,
            [pltpu.SemaphoreType.DMA] * NBUF,
        ],
        compiler_params=pltpu.CompilerParams(use_tc_tiling_on_sc=False),
    )
    def gather_kernel(idx_hbm, table_hbm, out_hbm, idx_v, rows_v,
                      sem_i, sem_g, sem_o):
        wid = lax.axis_index("s") * NC + lax.axis_index("c")
        base = wid * BPW

        def idx_copy(g, b):
            return pltpu.make_async_copy(
                idx_hbm.at[pl.ds(base + g * CHUNK, CHUNK)],
                idx_v.at[b], sem_i[b])

        def gather(b):
            return pltpu.make_async_copy(
                table_hbm.at[idx_v.at[b]], rows_v.at[b], sem_g[b])

        def out_store(g, b):
            return pltpu.make_async_copy(
                rows_v.at[b], out_hbm.at[pl.ds(base + g * CHUNK, CHUNK)],
                sem_o[b])

        for b in range(NBUF):
            idx_copy(b, b).start()

        @pl.loop(0, NCHUNK, step=NBUF)
        def _chunks(g0):
            for b in range(NBUF):
                g = g0 + b
                pb = (b - 1) % NBUF
                idx_copy(g, b).wait()

                @pl.when(g >= NBUF)
                def _():
                    out_store(g - NBUF, b).wait()

                gather(b).start()

                @pl.when(g >= 1)
                def _():
                    gather(pb).wait()

                    @pl.when(g - 1 + NBUF < NCHUNK)
                    def _():
                        idx_copy(g - 1 + NBUF, pb).start()

                    out_store(g - 1, pb).start()

        lb = (NCHUNK - 1) % NBUF
        gather(lb).wait()
        out_store(NCHUNK - 1, lb).start()
        for k in range(NBUF):
            g = NCHUNK - NBUF + k
            out_store(g, g % NBUF).wait()

    return gather_kernel


_gather = _make_gather()


@jax.jit
def kernel(x, W):
    xf = x.reshape(-1).astype(jnp.int32)
    out = _gather(xf, W)
    return out.reshape(BATCH, HIST, SIZE)


# speedup vs baseline: 2.5704x; 1.0004x over previous
import functools

import jax
import jax.numpy as jnp
from jax import lax
from jax.experimental import pallas as pl
from jax.experimental.pallas import tpu as pltpu
from jax.experimental.pallas import tpu_sc as plsc

BATCH = 16384
HIST = 200
SIZE = 16
B = BATCH * HIST
NC = 2
NS = 16
NW = NC * NS
BPW = B // NW
CHUNK = 1024
NCHUNK = BPW // CHUNK
NBUF = 4
SKEW = 2


def _make_gather():
    mesh = plsc.VectorSubcoreMesh(core_axis_name="c", subcore_axis_name="s")

    @functools.partial(
        pl.kernel,
        mesh=mesh,
        out_type=jax.ShapeDtypeStruct((B, SIZE), jnp.float32),
        scratch_types=[
            pltpu.VMEM((NBUF, CHUNK), jnp.int32),
            pltpu.VMEM((NBUF, CHUNK, SIZE), jnp.float32),
            [pltpu.SemaphoreType.DMA] * NBUF,
            [pltpu.SemaphoreType.DMA] * NBUF,
            [pltpu.SemaphoreType.DMA] * NBUF,
        ],
        compiler_params=pltpu.CompilerParams(use_tc_tiling_on_sc=False),
    )
    def gather_kernel(idx_hbm, table_hbm, out_hbm, idx_v, rows_v,
                      sem_i, sem_g, sem_o):
        wid = lax.axis_index("s") * NC + lax.axis_index("c")
        base = wid * BPW

        def idx_copy(g, b):
            return pltpu.make_async_copy(
                idx_hbm.at[pl.ds(base + g * CHUNK, CHUNK)],
                idx_v.at[b], sem_i[b])

        def gather(b):
            return pltpu.make_async_copy(
                table_hbm.at[idx_v.at[b]], rows_v.at[b], sem_g[b])

        def out_store(g, b):
            return pltpu.make_async_copy(
                rows_v.at[b], out_hbm.at[pl.ds(base + g * CHUNK, CHUNK)],
                sem_o[b])

        def retire(r, rb, prefetch):
            gather(rb).wait()
            if prefetch:
                @pl.when(r + NBUF < NCHUNK)
                def _():
                    idx_copy(r + NBUF, rb).start()
            out_store(r, rb).start()

        for b in range(NBUF):
            idx_copy(b, b).start()

        @pl.loop(0, NCHUNK, step=NBUF)
        def _chunks(g0):
            for b in range(NBUF):
                g = g0 + b
                rb = (b - SKEW) % NBUF
                idx_copy(g, b).wait()

                @pl.when(g >= NBUF)
                def _():
                    out_store(g - NBUF, b).wait()

                gather(b).start()

                @pl.when(g >= SKEW)
                def _():
                    retire(g - SKEW, rb, prefetch=True)

        for r in range(NCHUNK - SKEW, NCHUNK):
            retire(r, r % NBUF, prefetch=False)
        for g in range(NCHUNK - NBUF, NCHUNK):
            out_store(g, g % NBUF).wait()

    return gather_kernel


_gather = _make_gather()


@jax.jit
def kernel(x, W):
    xf = x.reshape(-1).astype(jnp.int32)
    out = _gather(xf, W)
    return out.reshape(BATCH, HIST, SIZE)


# speedup vs baseline: 2.5718x; 1.0005x over previous
import functools

import jax
import jax.numpy as jnp
from jax import lax
from jax.experimental import pallas as pl
from jax.experimental.pallas import tpu as pltpu
from jax.experimental.pallas import tpu_sc as plsc

BATCH = 16384
HIST = 200
SIZE = 16
NC = 2
NS = 16
NW = NC * NS
RPW = BATCH // NW
R = 8
NCHUNK = RPW // R
NBUF = 4
SKEW = 2


def _make_gather():
    mesh = plsc.VectorSubcoreMesh(core_axis_name="c", subcore_axis_name="s")

    @functools.partial(
        pl.kernel,
        mesh=mesh,
        out_type=jax.ShapeDtypeStruct((BATCH, HIST, SIZE), jnp.float32),
        scratch_types=[
            pltpu.VMEM((NBUF, R * HIST), jnp.int32),
            pltpu.VMEM((NBUF, R * HIST, SIZE), jnp.float32),
            [pltpu.SemaphoreType.DMA] * NBUF,
            [pltpu.SemaphoreType.DMA] * NBUF,
            [pltpu.SemaphoreType.DMA] * NBUF,
        ],
        compiler_params=pltpu.CompilerParams(use_tc_tiling_on_sc=False),
    )
    def gather_kernel(idx_hbm, table_hbm, out_hbm, idx_v, rows_v,
                      sem_i, sem_g, sem_o):
        wid = lax.axis_index("s") * NC + lax.axis_index("c")
        base = wid * RPW

        def idx_copies(g, b):
            return [pltpu.make_async_copy(
                        idx_hbm.at[base + g * R + r],
                        idx_v.at[b, pl.ds(r * HIST, HIST)], sem_i[b])
                    for r in range(R)]

        def gather(b):
            return pltpu.make_async_copy(
                table_hbm.at[idx_v.at[b]], rows_v.at[b], sem_g[b])

        def out_stores(g, b):
            return [pltpu.make_async_copy(
                        rows_v.at[b, pl.ds(r * HIST, HIST)],
                        out_hbm.at[base + g * R + r], sem_o[b])
                    for r in range(R)]

        def start(descs):
            for d in descs:
                d.start()

        def wait(descs):
            for d in descs:
                d.wait()

        def retire(rr, rb, prefetch):
            gather(rb).wait()
            if prefetch:
                @pl.when(rr + NBUF < NCHUNK)
                def _():
                    start(idx_copies(rr + NBUF, rb))
            start(out_stores(rr, rb))

        for b in range(NBUF):
            start(idx_copies(b, b))

        @pl.loop(0, NCHUNK, step=NBUF)
        def _chunks(g0):
            for b in range(NBUF):
                g = g0 + b
                rb = (b - SKEW) % NBUF
                wait(idx_copies(g, b))

                @pl.when(g >= NBUF)
                def _():
                    wait(out_stores(g - NBUF, b))

                gather(b).start()

                @pl.when(g >= SKEW)
                def _():
                    retire(g - SKEW, rb, prefetch=True)

        for rr in range(NCHUNK - SKEW, NCHUNK):
            retire(rr, rr % NBUF, prefetch=False)
        for g in range(NCHUNK - NBUF, NCHUNK):
            wait(out_stores(g, g % NBUF))

    return gather_kernel


_gather = _make_gather()


@jax.jit
def kernel(x, W):
    return _gather(x.astype(jnp.int32), W)


# speedup vs baseline: 4.7884x; 1.8619x over previous
import functools

import jax
import jax.numpy as jnp
from jax import lax
from jax.experimental import pallas as pl
from jax.experimental.pallas import tpu as pltpu
from jax.experimental.pallas import tpu_sc as plsc

BATCH = 16384
HIST = 200
SIZE = 16
NC = 2
NS = 16
NW = NC * NS
HG = HIST // 8
BG = BATCH // 128
NCH_TOT = HG * BG
CPW = NCH_TOT // NW
CL = 8 * 128
NBUF = 2
assert CPW % NBUF == 0


def _make_gather():
    mesh = plsc.VectorSubcoreMesh(core_axis_name="c", subcore_axis_name="s")

    @functools.partial(
        pl.kernel,
        mesh=mesh,
        out_type=jax.ShapeDtypeStruct((HIST, 2, BG, 8, 128), jnp.float32),
        scratch_types=[
            pltpu.VMEM((NBUF, CL), jnp.int32),
            pltpu.VMEM((NBUF, CL, SIZE), jnp.float32),
            pltpu.VMEM((NBUF, SIZE, CL), jnp.float32),
            [pltpu.SemaphoreType.DMA] * NBUF,
            [pltpu.SemaphoreType.DMA] * NBUF,
            [pltpu.SemaphoreType.DMA] * NBUF,
        ],
        compiler_params=pltpu.CompilerParams(
            use_tc_tiling_on_sc=False, needs_layout_passes=False),
    )
    def gather_kernel(idx_hbm, table_hbm, out_hbm, idx_v, rows_v, t_v,
                      sem_i, sem_g, sem_o):
        wid = lax.axis_index("s") * NC + lax.axis_index("c")
        c0 = wid * CPW

        iota = lax.iota(jnp.int32, 16)
        dcol = [jnp.full((16,), d, jnp.int32) for d in range(SIZE)]

        def idx_copies(c, b):
            hg = c // BG
            bg = lax.rem(c, BG)
            return [pltpu.make_async_copy(
                        idx_hbm.at[hg, bg, hl],
                        idx_v.at[b, pl.ds(hl * 128, 128)], sem_i[b])
                    for hl in range(8)]

        def gather(b):
            return pltpu.make_async_copy(
                table_hbm.at[idx_v.at[b]], rows_v.at[b], sem_g[b])

        def out_stores(c, b):
            hg = c // BG
            bg = lax.rem(c, BG)
            return [pltpu.make_async_copy(
                        t_v.at[b, pl.ds(dg * 8, 8), pl.ds(hl * 128, 128)],
                        out_hbm.at[hg * 8 + hl, dg, bg], sem_o[b])
                    for hl in range(8) for dg in range(2)]

        def start(ds):
            for d in ds:
                d.start()

        def wait(ds):
            for d in ds:
                d.wait()

        def transpose(b):
            rows = rows_v.at[b]

            @pl.loop(0, CL // 16)
            def _blk(blk):
                n0 = blk * 16
                ridx = iota + n0
                for d in range(SIZE):
                    v = plsc.load_gather(rows, [ridx, dcol[d]])
                    t_v[b, d, pl.ds(n0, 16)] = v

        def retire(c, b, prefetch):
            gather(b).wait()
            transpose(b)
            if prefetch:
                @pl.when(c + NBUF < CPW)
                def _():
                    start(idx_copies(c0 + c + NBUF, b))
            start(out_stores(c0 + c, b))

        for b in range(NBUF):
            start(idx_copies(c0 + b, b))

        @pl.loop(0, CPW, step=NBUF)
        def _chunks(g0):
            for b in range(NBUF):
                g = g0 + b
                pb = (b - 1) % NBUF
                wait(idx_copies(c0 + g, b))

                @pl.when(g >= NBUF)
                def _():
                    wait(out_stores(c0 + g - NBUF, b))

                gather(b).start()

                @pl.when(g >= 1)
                def _():
                    retire(g - 1, pb, prefetch=True)

        retire(CPW - 1, (CPW - 1) % NBUF, prefetch=False)
        for g in range(CPW - NBUF, CPW):
            wait(out_stores(c0 + g, g % NBUF))

    return gather_kernel


_gather = _make_gather()


@jax.jit
def kernel(x, W):
    x4 = (x.astype(jnp.int32).T
          .reshape(HG, 8, BG, 128).transpose(0, 2, 1, 3))
    out5 = _gather(x4, W)
    return (out5.transpose(0, 1, 3, 2, 4)
            .reshape(HIST, SIZE, BATCH).transpose(2, 0, 1))


# speedup vs baseline: 9.6292x; 2.0109x over previous
import functools

import jax
import jax.numpy as jnp
from jax import lax
from jax.experimental import pallas as pl
from jax.experimental.pallas import tpu as pltpu
from jax.experimental.pallas import tpu_sc as plsc

BATCH = 16384
HIST = 200
SIZE = 16
NC = 2
NS = 16
NW = NC * NS
HG = HIST // 8
BG = BATCH // 128
NCH_TOT = HG * BG
CPW = NCH_TOT // NW
CL = 8 * 128
NBUF = 2
assert CPW % NBUF == 0


def _make_gather():
    mesh = plsc.VectorSubcoreMesh(core_axis_name="c", subcore_axis_name="s")

    @functools.partial(
        pl.kernel,
        mesh=mesh,
        out_type=jax.ShapeDtypeStruct((HIST, 2, BG, 8, 128), jnp.float32),
        scratch_types=[
            pltpu.VMEM((NBUF, CL), jnp.int32),
            pltpu.VMEM((NBUF, CL, SIZE), jnp.float32),
            pltpu.VMEM((NBUF, SIZE, CL), jnp.float32),
            [pltpu.SemaphoreType.DMA] * NBUF,
            [pltpu.SemaphoreType.DMA] * NBUF,
            [pltpu.SemaphoreType.DMA] * NBUF,
        ],
        compiler_params=pltpu.CompilerParams(
            use_tc_tiling_on_sc=False, needs_layout_passes=False,
            disable_bounds_checks=True),
    )
    def gather_kernel(idx_hbm, table_hbm, out_hbm, idx_v, rows_v, t_v,
                      sem_i, sem_g, sem_o):
        wid = lax.axis_index("s") * NC + lax.axis_index("c")
        c0 = wid * CPW

        iota = lax.iota(jnp.int32, 16)
        dcol = [jnp.full((16,), d, jnp.int32) for d in range(SIZE)]

        def idx_copies(c, b):
            hg = c // BG
            bg = lax.rem(c, BG)
            return [pltpu.make_async_copy(
                        idx_hbm.at[hg, bg, hl],
                        idx_v.at[b, pl.ds(hl * 128, 128)], sem_i[b])
                    for hl in range(8)]

        def gather(b):
            return pltpu.make_async_copy(
                table_hbm.at[idx_v.at[b]], rows_v.at[b], sem_g[b])

        def out_stores(c, b):
            hg = c // BG
            bg = lax.rem(c, BG)
            return [pltpu.make_async_copy(
                        t_v.at[b, pl.ds(dg * 8, 8), pl.ds(hl * 128, 128)],
                        out_hbm.at[hg * 8 + hl, dg, bg], sem_o[b])
                    for hl in range(8) for dg in range(2)]

        def start(ds):
            for d in ds:
                d.start()

        def wait(ds):
            for d in ds:
                d.wait()

        def transpose(b):
            rows = rows_v.at[b]

            @functools.partial(plsc.parallel_loop, 0, CL // 16, unroll=4)
            def _blk(blk):
                n0 = blk * 16
                ridx = iota + n0
                for d in range(SIZE):
                    v = plsc.load_gather(rows, [ridx, dcol[d]])
                    t_v[b, d, pl.ds(n0, 16)] = v

        def retire(c, b, prefetch):
            gather(b).wait()
            transpose(b)
            if prefetch:
                @pl.when(c + NBUF < CPW)
                def _():
                    start(idx_copies(c0 + c + NBUF, b))
            start(out_stores(c0 + c, b))

        for b in range(NBUF):
            start(idx_copies(c0 + b, b))

        @pl.loop(0, CPW, step=NBUF)
        def _chunks(g0):
            for b in range(NBUF):
                g = g0 + b
                pb = (b - 1) % NBUF
                wait(idx_copies(c0 + g, b))

                @pl.when(g >= NBUF)
                def _():
                    wait(out_stores(c0 + g - NBUF, b))

                gather(b).start()

                @pl.when(g >= 1)
                def _():
                    retire(g - 1, pb, prefetch=True)

        retire(CPW - 1, (CPW - 1) % NBUF, prefetch=False)
        for g in range(CPW - NBUF, CPW):
            wait(out_stores(c0 + g, g % NBUF))

    return gather_kernel


_gather = _make_gather()


@jax.jit
def kernel(x, W):
    x4 = (x.astype(jnp.int32).T
          .reshape(HG, 8, BG, 128).transpose(0, 2, 1, 3))
    out5 = _gather(x4, W)
    return (out5.transpose(0, 1, 3, 2, 4)
            .reshape(HIST, SIZE, BATCH).transpose(2, 0, 1))
